# Initial kernel scaffold; baseline (speedup 1.0000x reference)
import functools

import jax
import jax.numpy as jnp
from jax import lax
from jax.experimental import pallas as pl
from jax.experimental.pallas import tpu as pltpu
from jax.experimental.pallas import tpu_sc as plsc

N = 10000
E = 160000
C = 16
SH = 9
NB = 8
RCUT = 5.0
D144 = C * SH

NC = 2
NS = 16
NW = NC * NS
KCH = 128
EP = 163840
RW = EP // NW
NCH = RW // KCH
EPR = EP // KCH
NPS = N // NS


def _mesh():
    return plsc.VectorSubcoreMesh(core_axis_name="c", subcore_axis_name="s")


def _gather_rows(table, idx2d, d):
    @functools.partial(
        pl.kernel,
        mesh=_mesh(),
        out_type=jax.ShapeDtypeStruct((EP, d), jnp.float32),
        scratch_types=[
            pltpu.VMEM((NCH, KCH), jnp.int32),
            pltpu.VMEM((KCH, d), jnp.float32),
            pltpu.SemaphoreType.DMA,
        ],
        name=f"sc_gather_rows_{d}",
    )
    def k(table_hbm, idx_hbm, out_hbm, idx_v, rows_v, sem):
        c = lax.axis_index("c")
        s = lax.axis_index("s")
        wid = s * NC + c
        pltpu.sync_copy(idx_hbm.at[pl.ds(wid * NCH, NCH)], idx_v)
        for j in range(NCH):
            pltpu.async_copy(table_hbm.at[idx_v.at[j]], rows_v, sem).wait()
            pltpu.sync_copy(rows_v, out_hbm.at[pl.ds(wid * RW + j * KCH, KCH)])

    return k(table, idx2d)


def _scatter_add(msg, idx2d, zeros_nd):
    @functools.partial(
        pl.kernel,
        mesh=_mesh(),
        out_type=jax.ShapeDtypeStruct((NC, N, D144), jnp.float32),
        scratch_types=[
            pltpu.VMEM((NCH, KCH), jnp.int32),
            pltpu.VMEM((KCH, D144), jnp.float32),
            pltpu.VMEM_SHARED((N, D144), jnp.float32),
        ],
        name="sc_scatter_add_144",
    )
    def k(msg_hbm, idx_hbm, zeros_hbm, out_hbm, idx_v, msg_v, acc_sh):
        c = lax.axis_index("c")
        s = lax.axis_index("s")
        wid = s * NC + c
        pltpu.sync_copy(zeros_hbm.at[pl.ds(s * NPS, NPS)],
                        acc_sh.at[pl.ds(s * NPS, NPS)])
        plsc.subcore_barrier()
        pltpu.sync_copy(idx_hbm.at[pl.ds(wid * NCH, NCH)], idx_v)
        for j in range(NCH):
            pltpu.sync_copy(msg_hbm.at[pl.ds(wid * RW + j * KCH, KCH)], msg_v)
            pltpu.sync_copy(msg_v, acc_sh.at[idx_v.at[j]], add=True)
        plsc.subcore_barrier()
        pltpu.sync_copy(acc_sh.at[pl.ds(s * NPS, NPS)],
                        out_hbm.at[c, pl.ds(s * NPS, NPS)])

    return k(msg, idx2d, zeros_nd)


_EBLK = 2048


def _radial_kernel(r):
    def body(r_ref, w0_ref, w1_ref, rad0_ref, rad1_ref):
        r = r_ref[...]
        n = lax.broadcasted_iota(jnp.float32, (1, NB), 1) + 1.0
        safe = jnp.where(r > 0.0, r, 1.0)
        rb = jnp.sqrt(2.0 / RCUT) * jnp.sin(n * (jnp.pi / RCUT) * safe) / safe
        rb = jnp.where(r > 0.0, rb, 0.0)
        rad0_ref[...] = jnp.dot(rb, w0_ref[...], preferred_element_type=jnp.float32)
        rad1_ref[...] = jnp.dot(rb, w1_ref[...], preferred_element_type=jnp.float32)

    return pl.pallas_call(
        body,
        grid=(EP // _EBLK,),
        in_specs=[
            pl.BlockSpec((_EBLK, 1), lambda i: (i, 0)),
            pl.BlockSpec((NB, C), lambda i: (0, 0)),
            pl.BlockSpec((NB, C), lambda i: (0, 0)),
        ],
        out_specs=[
            pl.BlockSpec((_EBLK, C), lambda i: (i, 0)),
            pl.BlockSpec((_EBLK, C), lambda i: (i, 0)),
        ],
        out_shape=[
            jax.ShapeDtypeStruct((EP, C), jnp.float32),
            jax.ShapeDtypeStruct((EP, C), jnp.float32),
        ],
    )(r)


def _node_embed_kernel(na, W_embed, W_sc0, W_sc1):
    def body(na_ref, we_ref, w0_ref, w1_ref, hs_ref, hsc0_ref, nsc1_ref):
        na = na_ref[...]
        hs = jnp.dot(na, we_ref[...], preferred_element_type=jnp.float32)
        hs_ref[...] = hs
        hsc0_ref[...] = hs * jnp.dot(na, w0_ref[...], preferred_element_type=jnp.float32)
        nsc1_ref[...] = jnp.dot(na, w1_ref[...], preferred_element_type=jnp.float32)

    return pl.pallas_call(
        body,
        out_shape=[
            jax.ShapeDtypeStruct((N, C), jnp.float32),
            jax.ShapeDtypeStruct((N, C), jnp.float32),
            jax.ShapeDtypeStruct((N, C), jnp.float32),
        ],
    )(na, W_embed, W_sc0, W_sc1)


def _msg0_kernel(radial0, s0, sh):
    def body(rad_ref, s_ref, sh_ref, out_ref):
        a = rad_ref[...] * s_ref[...]
        sh = sh_ref[...]
        out_ref[...] = jnp.concatenate(
            [a * sh[:, d:d + 1] for d in range(SH)], axis=1)

    return pl.pallas_call(
        body,
        grid=(EP // _EBLK,),
        in_specs=[
            pl.BlockSpec((_EBLK, C), lambda i: (i, 0)),
            pl.BlockSpec((_EBLK, C), lambda i: (i, 0)),
            pl.BlockSpec((_EBLK, SH), lambda i: (i, 0)),
        ],
        out_specs=pl.BlockSpec((_EBLK, D144), lambda i: (i, 0)),
        out_shape=jax.ShapeDtypeStruct((EP, D144), jnp.float32),
    )(radial0, s0, sh)


def _msg1_kernel(g, radial1, sh):
    def body(g_ref, rad_ref, sh_ref, out_ref):
        g = g_ref[...]
        sh = sh_ref[...]
        s1 = g[:, 0:C] * sh[:, 0:1]
        for d in range(1, SH):
            s1 = s1 + g[:, d * C:(d + 1) * C] * sh[:, d:d + 1]
        a = rad_ref[...] * s1
        out_ref[...] = jnp.concatenate(
            [a * sh[:, d:d + 1] for d in range(SH)], axis=1)

    return pl.pallas_call(
        body,
        grid=(EP // _EBLK,),
        in_specs=[
            pl.BlockSpec((_EBLK, D144), lambda i: (i, 0)),
            pl.BlockSpec((_EBLK, C), lambda i: (i, 0)),
            pl.BlockSpec((_EBLK, SH), lambda i: (i, 0)),
        ],
        out_specs=pl.BlockSpec((_EBLK, D144), lambda i: (i, 0)),
        out_shape=jax.ShapeDtypeStruct((EP, D144), jnp.float32),
    )(g, radial1, sh)


def _combine_kernel(agg2, sc_d0, Wp, Wrs, Wrl2, want_h):
    def body(agg_ref, sc_ref, wp_ref, wrs_ref, wrl2_ref, *outs):
        agg = agg_ref[0] + agg_ref[1]
        ad = [agg[:, d * C:(d + 1) * C] for d in range(SH)]
        nrm = ad[0] * ad[0]
        for d in range(1, SH):
            nrm = nrm + ad[d] * ad[d]
        w0 = wp_ref[0:1, :]
        w1 = wp_ref[1:2, :]
        w2 = wp_ref[2:3, :]
        a0 = ad[0]
        h = [w0 * ad[d] + w1 * ad[d] * a0 + w2 * ad[d] * nrm for d in range(SH)]
        h[0] = h[0] + sc_ref[...]
        r_scalar = jnp.sum(h[0] * wrs_ref[...])
        rl2 = [jnp.sum(h[4 + j] * wrl2_ref[...]) for j in range(5)]
        pr = jnp.concatenate(
            [r_scalar.reshape(1, 1)] + [v.reshape(1, 1) for v in rl2]
            + [jnp.zeros((1, 2), jnp.float32)], axis=1)
        if want_h:
            outs[0][...] = jnp.concatenate(h, axis=1)
            outs[1][...] = pr
        else:
            outs[0][...] = pr

    out_shape = [jax.ShapeDtypeStruct((1, 8), jnp.float32)]
    if want_h:
        out_shape = [jax.ShapeDtypeStruct((N, D144), jnp.float32)] + out_shape
    return pl.pallas_call(body, out_shape=out_shape)(agg2, sc_d0, Wp, Wrs, Wrl2)


def _mul_kernel(x, y):
    def body(x_ref, y_ref, o_ref):
        o_ref[...] = x_ref[...] * y_ref[...]
    return pl.pallas_call(
        body, out_shape=jax.ShapeDtypeStruct(x.shape, jnp.float32))(x, y)


def _forward(r, sh, na, src, dst, W_embed, W_rad, W_sc, W_prod, W_rs, W_rl2):
    pad = EP - E
    r_p = jnp.pad(r, (0, pad)).reshape(EP, 1)
    sh_p = jnp.pad(sh, ((0, pad), (0, 0)))
    src2d = jnp.pad(src, (0, pad)).reshape(EPR, KCH)
    dst2d = jnp.pad(dst, (0, pad)).reshape(EPR, KCH)
    zeros_nd = jnp.zeros((N, D144), jnp.float32)

    radial0, radial1 = _radial_kernel(r_p)
    h_scalar, hsc0, nsc1 = _node_embed_kernel(na, W_embed, W_sc[0], W_sc[1])

    s0 = _gather_rows(h_scalar, src2d, C)
    msg0 = _msg0_kernel(radial0, s0, sh_p)
    agg0_2 = _scatter_add(msg0, dst2d, zeros_nd)
    h0, pr0 = _combine_kernel(agg0_2, hsc0, W_prod[0], W_rs[0:1], W_rl2[0:1], True)

    g1 = _gather_rows(h0, src2d, D144)
    msg1 = _msg1_kernel(g1, radial1, sh_p)
    agg1_2 = _scatter_add(msg1, dst2d, zeros_nd)
    sc1 = _mul_kernel(h0[:, 0:C], nsc1)
    (pr1,) = _combine_kernel(agg1_2, sc1, W_prod[1], W_rs[1:2], W_rl2[1:2], False)

    return (pr0 + pr1)[0, :6]


def kernel(x, x_v, node_attr, edge_index, W_embed, W_rad, W_sc, W_prod, W_rs, W_rl2):
    outs = []
    for b in range(x.shape[0]):
        outs.append(_forward(x[b], x_v[b], node_attr[b],
                             edge_index[b, 0], edge_index[b, 1],
                             W_embed, W_rad, W_sc, W_prod, W_rs, W_rl2))
    return jnp.stack(outs, axis=0)


# speedup vs baseline: 15.8088x; 15.8088x over previous
import functools

import jax
import jax.numpy as jnp
from jax import lax
from jax.experimental import pallas as pl
from jax.experimental.pallas import tpu as pltpu
from jax.experimental.pallas import tpu_sc as plsc

N = 10000
E = 160000
C = 16
SH = 9
NB = 8
RCUT = 5.0
D144 = C * SH

NC = 2
NS = 16
NW = NC * NS
KCH = 128
EP = 163840
RW = EP // NW
NCH = RW // KCH
EPR = EP // KCH
NPS = N // NS


def _mesh():
    return plsc.VectorSubcoreMesh(core_axis_name="c", subcore_axis_name="s")


def _gather_rows(table, idx2d, d):
    @functools.partial(
        pl.kernel,
        mesh=_mesh(),
        out_type=jax.ShapeDtypeStruct((EP, d), jnp.float32),
        scratch_types=[
            pltpu.VMEM((NCH, KCH), jnp.int32),
            pltpu.VMEM((KCH, d), jnp.float32),
            pltpu.SemaphoreType.DMA,
        ],
        name=f"sc_gather_rows_{d}",
        compiler_params=pltpu.CompilerParams(use_tc_tiling_on_sc=False),
    )
    def k(table_hbm, idx_hbm, out_hbm, idx_v, rows_v, sem):
        c = lax.axis_index("c")
        s = lax.axis_index("s")
        wid = s * NC + c
        pltpu.sync_copy(idx_hbm.at[pl.ds(wid * NCH, NCH)], idx_v)
        for j in range(NCH):
            pltpu.async_copy(table_hbm.at[idx_v.at[j]], rows_v, sem).wait()
            pltpu.sync_copy(rows_v, out_hbm.at[pl.ds(wid * RW + j * KCH, KCH)])

    return k(table, idx2d)


def _scatter_add(msg, idx2d, zeros_nd):
    @functools.partial(
        pl.kernel,
        mesh=_mesh(),
        out_type=jax.ShapeDtypeStruct((NC, N, D144), jnp.float32),
        scratch_types=[
            pltpu.VMEM((NCH, KCH), jnp.int32),
            pltpu.VMEM((KCH, D144), jnp.float32),
            pltpu.VMEM_SHARED((N, D144), jnp.float32),
        ],
        name="sc_scatter_add_144",
        compiler_params=pltpu.CompilerParams(use_tc_tiling_on_sc=False),
    )
    def k(msg_hbm, idx_hbm, zeros_hbm, out_hbm, idx_v, msg_v, acc_sh):
        c = lax.axis_index("c")
        s = lax.axis_index("s")
        wid = s * NC + c
        pltpu.sync_copy(zeros_hbm.at[pl.ds(s * NPS, NPS)],
                        acc_sh.at[pl.ds(s * NPS, NPS)])
        plsc.subcore_barrier()
        pltpu.sync_copy(idx_hbm.at[pl.ds(wid * NCH, NCH)], idx_v)
        for j in range(NCH):
            pltpu.sync_copy(msg_hbm.at[pl.ds(wid * RW + j * KCH, KCH)], msg_v)
            pltpu.sync_copy(msg_v, acc_sh.at[idx_v.at[j]], add=True)
        plsc.subcore_barrier()
        pltpu.sync_copy(acc_sh.at[pl.ds(s * NPS, NPS)],
                        out_hbm.at[c, pl.ds(s * NPS, NPS)])

    return k(msg, idx2d, zeros_nd)


_EBLK = 2048


def _radial_kernel(r, w0, w1):
    def body(r_ref, w0_ref, w1_ref, rad0_ref, rad1_ref):
        r = r_ref[...]
        n = lax.broadcasted_iota(jnp.int32, (1, NB), 1).astype(jnp.float32) + 1.0
        safe = jnp.where(r > 0.0, r, 1.0)
        rb = jnp.sqrt(2.0 / RCUT) * jnp.sin(n * (jnp.pi / RCUT) * safe) / safe
        rb = jnp.where(r > 0.0, rb, 0.0)
        rad0_ref[...] = jnp.dot(rb, w0_ref[...], preferred_element_type=jnp.float32)
        rad1_ref[...] = jnp.dot(rb, w1_ref[...], preferred_element_type=jnp.float32)

    return pl.pallas_call(
        body,
        grid=(EP // _EBLK,),
        in_specs=[
            pl.BlockSpec((_EBLK, 1), lambda i: (i, 0)),
            pl.BlockSpec((NB, C), lambda i: (0, 0)),
            pl.BlockSpec((NB, C), lambda i: (0, 0)),
        ],
        out_specs=[
            pl.BlockSpec((_EBLK, C), lambda i: (i, 0)),
            pl.BlockSpec((_EBLK, C), lambda i: (i, 0)),
        ],
        out_shape=[
            jax.ShapeDtypeStruct((EP, C), jnp.float32),
            jax.ShapeDtypeStruct((EP, C), jnp.float32),
        ],
    )(r, w0, w1)


def _node_embed_kernel(na, W_embed, W_sc0, W_sc1):
    def body(na_ref, we_ref, w0_ref, w1_ref, hs_ref, hsc0_ref, nsc1_ref):
        na = na_ref[...]
        hs = jnp.dot(na, we_ref[...], preferred_element_type=jnp.float32)
        hs_ref[...] = hs
        hsc0_ref[...] = hs * jnp.dot(na, w0_ref[...], preferred_element_type=jnp.float32)
        nsc1_ref[...] = jnp.dot(na, w1_ref[...], preferred_element_type=jnp.float32)

    return pl.pallas_call(
        body,
        out_shape=[
            jax.ShapeDtypeStruct((N, C), jnp.float32),
            jax.ShapeDtypeStruct((N, C), jnp.float32),
            jax.ShapeDtypeStruct((N, C), jnp.float32),
        ],
    )(na, W_embed, W_sc0, W_sc1)


def _msg0_kernel(radial0, s0, sh):
    def body(rad_ref, s_ref, sh_ref, out_ref):
        a = rad_ref[...] * s_ref[...]
        sh = sh_ref[...]
        out_ref[...] = jnp.concatenate(
            [a * sh[:, d:d + 1] for d in range(SH)], axis=1)

    return pl.pallas_call(
        body,
        grid=(EP // _EBLK,),
        in_specs=[
            pl.BlockSpec((_EBLK, C), lambda i: (i, 0)),
            pl.BlockSpec((_EBLK, C), lambda i: (i, 0)),
            pl.BlockSpec((_EBLK, SH), lambda i: (i, 0)),
        ],
        out_specs=pl.BlockSpec((_EBLK, D144), lambda i: (i, 0)),
        out_shape=jax.ShapeDtypeStruct((EP, D144), jnp.float32),
    )(radial0, s0, sh)


def _msg1_kernel(g, radial1, sh):
    def body(g_ref, rad_ref, sh_ref, out_ref):
        g = g_ref[...]
        sh = sh_ref[...]
        s1 = g[:, 0:C] * sh[:, 0:1]
        for d in range(1, SH):
            s1 = s1 + g[:, d * C:(d + 1) * C] * sh[:, d:d + 1]
        a = rad_ref[...] * s1
        out_ref[...] = jnp.concatenate(
            [a * sh[:, d:d + 1] for d in range(SH)], axis=1)

    return pl.pallas_call(
        body,
        grid=(EP // _EBLK,),
        in_specs=[
            pl.BlockSpec((_EBLK, D144), lambda i: (i, 0)),
            pl.BlockSpec((_EBLK, C), lambda i: (i, 0)),
            pl.BlockSpec((_EBLK, SH), lambda i: (i, 0)),
        ],
        out_specs=pl.BlockSpec((_EBLK, D144), lambda i: (i, 0)),
        out_shape=jax.ShapeDtypeStruct((EP, D144), jnp.float32),
    )(g, radial1, sh)


def _combine_kernel(agg2, sc_d0, Wp, Wrs, Wrl2, want_h):
    NBLK = 1000

    def body(agg_ref, sc_ref, wp_ref, wrs_ref, wrl2_ref, *outs):
        agg = agg_ref[0] + agg_ref[1]
        ad = [agg[:, d * C:(d + 1) * C] for d in range(SH)]
        nrm = ad[0] * ad[0]
        for d in range(1, SH):
            nrm = nrm + ad[d] * ad[d]
        w0 = wp_ref[0:1, :]
        w1 = wp_ref[1:2, :]
        w2 = wp_ref[2:3, :]
        a0 = ad[0]
        h = [w0 * ad[d] + w1 * ad[d] * a0 + w2 * ad[d] * nrm for d in range(SH)]
        h[0] = h[0] + sc_ref[...]
        r_scalar = jnp.sum(h[0] * wrs_ref[...])
        rl2 = [jnp.sum(h[4 + j] * wrl2_ref[...]) for j in range(5)]
        pr = jnp.concatenate(
            [r_scalar.reshape(1, 1)] + [v.reshape(1, 1) for v in rl2]
            + [jnp.zeros((1, 2), jnp.float32)], axis=1)
        pr_ref = outs[-1]
        i = pl.program_id(0)

        @pl.when(i == 0)
        def _():
            pr_ref[...] = jnp.zeros((1, 8), jnp.float32)

        pr_ref[...] += pr
        if want_h:
            outs[0][...] = jnp.concatenate(h, axis=1)

    out_shape = [jax.ShapeDtypeStruct((1, 8), jnp.float32)]
    out_specs = [pl.BlockSpec((1, 8), lambda i: (0, 0))]
    if want_h:
        out_shape = [jax.ShapeDtypeStruct((N, D144), jnp.float32)] + out_shape
        out_specs = [pl.BlockSpec((NBLK, D144), lambda i: (i, 0))] + out_specs
    return pl.pallas_call(
        body,
        grid=(N // NBLK,),
        in_specs=[
            pl.BlockSpec((2, NBLK, D144), lambda i: (0, i, 0)),
            pl.BlockSpec((NBLK, C), lambda i: (i, 0)),
            pl.BlockSpec((3, C), lambda i: (0, 0)),
            pl.BlockSpec((1, C), lambda i: (0, 0)),
            pl.BlockSpec((1, C), lambda i: (0, 0)),
        ],
        out_specs=out_specs,
        out_shape=out_shape,
    )(agg2, sc_d0, Wp, Wrs, Wrl2)


def _mul_kernel(x, y):
    def body(x_ref, y_ref, o_ref):
        o_ref[...] = x_ref[...] * y_ref[...]
    return pl.pallas_call(
        body, out_shape=jax.ShapeDtypeStruct(x.shape, jnp.float32))(x, y)


def _forward(r, sh, na, src, dst, W_embed, W_rad, W_sc, W_prod, W_rs, W_rl2):
    pad = EP - E
    r_p = jnp.pad(r, (0, pad)).reshape(EP, 1)
    sh_p = jnp.pad(sh, ((0, pad), (0, 0)))
    src2d = jnp.pad(src, (0, pad)).reshape(EPR, KCH)
    dst2d = jnp.pad(dst, (0, pad)).reshape(EPR, KCH)
    zeros_nd = jnp.zeros((N, D144), jnp.float32)

    radial0, radial1 = _radial_kernel(r_p, W_rad[0], W_rad[1])
    h_scalar, hsc0, nsc1 = _node_embed_kernel(na, W_embed, W_sc[0], W_sc[1])

    s0 = _gather_rows(h_scalar, src2d, C)
    msg0 = _msg0_kernel(radial0, s0, sh_p)
    agg0_2 = _scatter_add(msg0, dst2d, zeros_nd)
    h0, pr0 = _combine_kernel(agg0_2, hsc0, W_prod[0], W_rs[0:1], W_rl2[0:1], True)

    g1 = _gather_rows(h0, src2d, D144)
    msg1 = _msg1_kernel(g1, radial1, sh_p)
    agg1_2 = _scatter_add(msg1, dst2d, zeros_nd)
    sc1 = _mul_kernel(h0[:, 0:C], nsc1)
    (pr1,) = _combine_kernel(agg1_2, sc1, W_prod[1], W_rs[1:2], W_rl2[1:2], False)

    return (pr0 + pr1)[0, :6]


def kernel(x, x_v, node_attr, edge_index, W_embed, W_rad, W_sc, W_prod, W_rs, W_rl2):
    outs = []
    for b in range(x.shape[0]):
        outs.append(_forward(x[b], x_v[b], node_attr[b],
                             edge_index[b, 0], edge_index[b, 1],
                             W_embed, W_rad, W_sc, W_prod, W_rs, W_rl2))
    return jnp.stack(outs, axis=0)


# speedup vs baseline: 40.2172x; 2.5440x over previous
import functools

import jax
import jax.numpy as jnp
from jax import lax
from jax.experimental import pallas as pl
from jax.experimental.pallas import tpu as pltpu
from jax.experimental.pallas import tpu_sc as plsc

N = 10000
E = 160000
C = 16
SH = 9
NB = 8
RCUT = 5.0
D144 = C * SH

NC = 2
NS = 16
NW = NC * NS
KCH = 80
EP = 163840
RW = EP // NW
NCH = RW // KCH
EPR = EP // KCH
NPS = N // NS
IR = 4
NSUP = NCH // IR


def _mesh():
    return plsc.VectorSubcoreMesh(core_axis_name="c", subcore_axis_name="s")


def _sc_layer(table, radial, sh16, idx_il, zeros_nd, layer):
    D = C if layer == 0 else D144

    def compute_chunk(g_b, r_b, s_b, m_b):
        def edge(e, _):
            srow = s_b[e]
            if layer == 0:
                arow = r_b[e] * g_b[e]
            else:
                s1 = g_b[e, pl.ds(0, C)] * srow[0]
                for d in range(1, SH):
                    s1 = s1 + g_b[e, pl.ds(d * C, C)] * srow[d]
                arow = r_b[e] * s1
            for d in range(SH):
                m_b[e, pl.ds(d * C, C)] = arow * srow[d]
            return _

        lax.fori_loop(0, KCH, edge, None)

    @functools.partial(
        pl.kernel,
        mesh=_mesh(),
        out_type=jax.ShapeDtypeStruct((NC, N, D144), jnp.float32),
        scratch_types=[
            pltpu.VMEM((2, 2 * IR, KCH), jnp.int32),
            pltpu.VMEM((2, KCH, D), jnp.float32),
            pltpu.VMEM((2, KCH, C), jnp.float32),
            pltpu.VMEM((2, KCH, C), jnp.float32),
            pltpu.VMEM((KCH, D144), jnp.float32),
            pltpu.VMEM_SHARED((N, D144), jnp.float32),
            pltpu.SemaphoreType.DMA,
            pltpu.SemaphoreType.DMA,
            pltpu.SemaphoreType.DMA,
        ],
        name=f"sc_layer{layer}",
        compiler_params=pltpu.CompilerParams(use_tc_tiling_on_sc=False),
    )
    def k(table_hbm, rad_hbm, sh_hbm, idx_hbm, zeros_hbm, out_hbm,
          iring, gbuf, rbuf, sbuf, mbuf, acc_sh, isem, gsem, lsem):
        c = lax.axis_index("c")
        s = lax.axis_index("s")
        wid = s * NC + c
        base = wid * RW
        irow0 = wid * NCH * 2
        pltpu.sync_copy(zeros_hbm.at[pl.ds(s * NPS, NPS)],
                        acc_sh.at[pl.ds(s * NPS, NPS)])
        plsc.subcore_barrier()

        def start(jj, b, src_row):
            pltpu.async_copy(table_hbm.at[src_row], gbuf.at[b], gsem)
            pltpu.async_copy(rad_hbm.at[pl.ds(base + jj * KCH, KCH)],
                             rbuf.at[b], lsem)
            pltpu.async_copy(sh_hbm.at[pl.ds(base + jj * KCH, KCH)],
                             sbuf.at[b], lsem)

        def wait(b):
            pltpu.make_async_copy(table_hbm.at[pl.ds(0, KCH)], gbuf.at[b],
                                  gsem).wait()
            pltpu.make_async_copy(rad_hbm.at[pl.ds(0, KCH)], rbuf.at[b],
                                  lsem).wait()
            pltpu.make_async_copy(sh_hbm.at[pl.ds(0, KCH)], sbuf.at[b],
                                  lsem).wait()

        pltpu.sync_copy(idx_hbm.at[pl.ds(irow0, 2 * IR)], iring.at[0])
        pltpu.async_copy(idx_hbm.at[pl.ds(irow0 + 2 * IR, 2 * IR)],
                         iring.at[1], isem)
        start(0, 0, iring.at[0, 0])
        start(1, 1, iring.at[0, 2])

        def super_step(u, _):
            slot = u % 2
            nslot = (u + 1) % 2

            @pl.when(u > 0)
            def _():
                pltpu.make_async_copy(idx_hbm.at[pl.ds(0, 2 * IR)],
                                      iring.at[0], isem).wait()

            for kk in range(IR):
                b = kk % 2
                jj = u * IR + kk
                wait(b)
                compute_chunk(gbuf.at[b], rbuf.at[b], sbuf.at[b], mbuf)

                @pl.when(jj + 2 < NCH)
                def _():
                    if kk < IR - 2:
                        src_row = iring.at[slot, 2 * (kk + 2)]
                    else:
                        src_row = iring.at[nslot, 2 * (kk + 2 - IR)]
                    start(jj + 2, b, src_row)

                pltpu.sync_copy(mbuf, acc_sh.at[iring.at[slot, 2 * kk + 1]],
                                add=True)

            @pl.when(u + 2 < NSUP)
            def _():
                pltpu.async_copy(
                    idx_hbm.at[pl.ds(irow0 + (u + 2) * 2 * IR, 2 * IR)],
                    iring.at[slot], isem)
            return _

        lax.fori_loop(0, NSUP, super_step, None)
        plsc.subcore_barrier()
        pltpu.sync_copy(acc_sh.at[pl.ds(s * NPS, NPS)],
                        out_hbm.at[c, pl.ds(s * NPS, NPS)])

    return k(table, radial, sh16, idx_il, zeros_nd)


_EBLK = 2048


def _radial_kernel(r, w0, w1):
    def body(r_ref, w0_ref, w1_ref, rad0_ref, rad1_ref):
        r = r_ref[...]
        n = lax.broadcasted_iota(jnp.int32, (1, NB), 1).astype(jnp.float32) + 1.0
        safe = jnp.where(r > 0.0, r, 1.0)
        rb = jnp.sqrt(2.0 / RCUT) * jnp.sin(n * (jnp.pi / RCUT) * safe) / safe
        rb = jnp.where(r > 0.0, rb, 0.0)
        rad0_ref[...] = jnp.dot(rb, w0_ref[...], preferred_element_type=jnp.float32)
        rad1_ref[...] = jnp.dot(rb, w1_ref[...], preferred_element_type=jnp.float32)

    return pl.pallas_call(
        body,
        grid=(EP // _EBLK,),
        in_specs=[
            pl.BlockSpec((_EBLK, 1), lambda i: (i, 0)),
            pl.BlockSpec((NB, C), lambda i: (0, 0)),
            pl.BlockSpec((NB, C), lambda i: (0, 0)),
        ],
        out_specs=[
            pl.BlockSpec((_EBLK, C), lambda i: (i, 0)),
            pl.BlockSpec((_EBLK, C), lambda i: (i, 0)),
        ],
        out_shape=[
            jax.ShapeDtypeStruct((EP, C), jnp.float32),
            jax.ShapeDtypeStruct((EP, C), jnp.float32),
        ],
    )(r, w0, w1)


def _node_embed_kernel(na, W_embed, W_sc0, W_sc1):
    def body(na_ref, we_ref, w0_ref, w1_ref, hs_ref, hsc0_ref, nsc1_ref):
        na = na_ref[...]
        hs = jnp.dot(na, we_ref[...], preferred_element_type=jnp.float32)
        hs_ref[...] = hs
        hsc0_ref[...] = hs * jnp.dot(na, w0_ref[...], preferred_element_type=jnp.float32)
        nsc1_ref[...] = jnp.dot(na, w1_ref[...], preferred_element_type=jnp.float32)

    return pl.pallas_call(
        body,
        out_shape=[
            jax.ShapeDtypeStruct((N, C), jnp.float32),
            jax.ShapeDtypeStruct((N, C), jnp.float32),
            jax.ShapeDtypeStruct((N, C), jnp.float32),
        ],
    )(na, W_embed, W_sc0, W_sc1)


def _combine_kernel(agg2, sc_d0, consts, want_h):
    S, S0, P, R, w0b, w1b, w2b = consts
    NBLK = 1000

    def body(agg_ref, sc_ref, s_ref, s0_ref, p_ref, r_ref, w0_ref, w1_ref,
             w2_ref, *outs):
        agg = agg_ref[0] + agg_ref[1]
        nrmb = jnp.dot(agg * agg, s_ref[...], preferred_element_type=jnp.float32)
        a0b = jnp.dot(agg, s0_ref[...], preferred_element_type=jnp.float32)
        scb = jnp.dot(sc_ref[...], p_ref[...], preferred_element_type=jnp.float32)
        h = (w0_ref[...] * agg + w1_ref[...] * agg * a0b
             + w2_ref[...] * agg * nrmb + scb)
        pr = jnp.sum(jnp.dot(h, r_ref[...], preferred_element_type=jnp.float32),
                     axis=0, keepdims=True)
        pr_ref = outs[-1]
        i = pl.program_id(0)

        @pl.when(i == 0)
        def _():
            pr_ref[...] = jnp.zeros((1, 8), jnp.float32)

        pr_ref[...] += pr
        if want_h:
            outs[0][...] = h

    out_shape = [jax.ShapeDtypeStruct((1, 8), jnp.float32)]
    out_specs = [pl.BlockSpec((1, 8), lambda i: (0, 0))]
    if want_h:
        out_shape = [jax.ShapeDtypeStruct((N, D144), jnp.float32)] + out_shape
        out_specs = [pl.BlockSpec((NBLK, D144), lambda i: (i, 0))] + out_specs
    return pl.pallas_call(
        body,
        grid=(N // NBLK,),
        in_specs=[
            pl.BlockSpec((2, NBLK, D144), lambda i: (0, i, 0)),
            pl.BlockSpec((NBLK, C), lambda i: (i, 0)),
            pl.BlockSpec((D144, D144), lambda i: (0, 0)),
            pl.BlockSpec((D144, D144), lambda i: (0, 0)),
            pl.BlockSpec((C, D144), lambda i: (0, 0)),
            pl.BlockSpec((D144, 8), lambda i: (0, 0)),
            pl.BlockSpec((1, D144), lambda i: (0, 0)),
            pl.BlockSpec((1, D144), lambda i: (0, 0)),
            pl.BlockSpec((1, D144), lambda i: (0, 0)),
        ],
        out_specs=out_specs,
        out_shape=out_shape,
    )(agg2, sc_d0, S, S0, P, R, w0b, w1b, w2b)


def _mul_kernel(x, y):
    def body(x_ref, y_ref, o_ref):
        o_ref[...] = x_ref[...] * y_ref[...]
    return pl.pallas_call(
        body, out_shape=jax.ShapeDtypeStruct(x.shape, jnp.float32))(x, y)


def _layer_consts(W_prod_l, W_rs_l, W_rl2_l):
    P = jnp.tile(jnp.eye(C, dtype=jnp.float32), (1, SH))
    S = P.T @ P
    S0 = jnp.concatenate([P, jnp.zeros((D144 - C, D144), jnp.float32)], axis=0)
    P0 = jnp.concatenate([jnp.eye(C, dtype=jnp.float32),
                          jnp.zeros((C, D144 - C), jnp.float32)], axis=1)
    R = jnp.zeros((D144, 8), jnp.float32)
    R = R.at[0:C, 0].set(W_rs_l)
    for j in range(5):
        R = R.at[(4 + j) * C:(5 + j) * C, 1 + j].set(W_rl2_l)
    w0b = jnp.tile(W_prod_l[0][None, :], (1, SH))
    w1b = jnp.tile(W_prod_l[1][None, :], (1, SH))
    w2b = jnp.tile(W_prod_l[2][None, :], (1, SH))
    return S, S0, P0, R, w0b, w1b, w2b


def _forward(r, sh, na, src, dst, W_embed, W_rad, W_sc, W_prod, W_rs, W_rl2):
    pad = EP - E
    r_p = jnp.pad(r, (0, pad)).reshape(EP, 1)
    sh16 = jnp.pad(sh, ((0, pad), (0, C - SH)))
    src2d = jnp.pad(src, (0, pad)).reshape(EPR, KCH)
    dst2d = jnp.pad(dst, (0, pad)).reshape(EPR, KCH)
    idx_il = jnp.stack([src2d, dst2d], axis=1).reshape(2 * EPR, KCH)
    zeros_nd = jnp.zeros((N, D144), jnp.float32)

    radial0, radial1 = _radial_kernel(r_p, W_rad[0], W_rad[1])
    h_scalar, hsc0, nsc1 = _node_embed_kernel(na, W_embed, W_sc[0], W_sc[1])

    agg0_2 = _sc_layer(h_scalar, radial0, sh16, idx_il, zeros_nd, 0)
    c0 = _layer_consts(W_prod[0], W_rs[0], W_rl2[0])
    h0, pr0 = _combine_kernel(agg0_2, hsc0, c0, True)

    agg1_2 = _sc_layer(h0, radial1, sh16, idx_il, zeros_nd, 1)
    sc1 = _mul_kernel(h0[:, 0:C], nsc1)
    c1 = _layer_consts(W_prod[1], W_rs[1], W_rl2[1])
    (pr1,) = _combine_kernel(agg1_2, sc1, c1, False)

    return (pr0 + pr1)[0, :6]


def kernel(x, x_v, node_attr, edge_index, W_embed, W_rad, W_sc, W_prod, W_rs, W_rl2):
    outs = []
    for b in range(x.shape[0]):
        outs.append(_forward(x[b], x_v[b], node_attr[b],
                             edge_index[b, 0], edge_index[b, 1],
                             W_embed, W_rad, W_sc, W_prod, W_rs, W_rl2))
    return jnp.stack(outs, axis=0)


# speedup vs baseline: 55.0410x; 1.3686x over previous
import functools

import jax
import jax.numpy as jnp
from jax import lax
from jax.experimental import pallas as pl
from jax.experimental.pallas import tpu as pltpu
from jax.experimental.pallas import tpu_sc as plsc

N = 10000
E = 160000
C = 16
SH = 9
NB = 8
RCUT = 5.0
D144 = C * SH

NC = 2
NS = 16
NW = NC * NS
KCH = 80
EP = 163840
RW = EP // NW
NCH = RW // KCH
EPR = EP // KCH
NPS = N // NS
IR = 4
NSUP = NCH // IR


def _mesh():
    return plsc.VectorSubcoreMesh(core_axis_name="c", subcore_axis_name="s")


def _sc_layer(table, eb, idx_il, zeros_nd, layer):
    D = C if layer == 0 else D144
    ECOL = 0 if layer == 0 else C

    def compute_chunk(g_b, e_b, m_b):
        def edge(e, _):
            if layer == 0:
                arow = e_b[e, pl.ds(0, C)] * g_b[e]
                srow = e_b[e, pl.ds(C, C)]
            else:
                srow = e_b[e, pl.ds(0, C)]
                s1 = g_b[e, pl.ds(0, C)] * srow[0]
                for d in range(1, SH):
                    s1 = s1 + g_b[e, pl.ds(d * C, C)] * srow[d]
                arow = e_b[e, pl.ds(C, C)] * s1
            for d in range(SH):
                m_b[e, pl.ds(d * C, C)] = arow * srow[d]
            return _

        lax.fori_loop(0, KCH, edge, None)

    @functools.partial(
        pl.kernel,
        mesh=_mesh(),
        out_type=jax.ShapeDtypeStruct((NC, N, D144), jnp.float32),
        scratch_types=[
            pltpu.VMEM((2, 2 * IR, KCH), jnp.int32),
            pltpu.VMEM((2, KCH, D), jnp.float32),
            pltpu.VMEM((2, KCH, 2 * C), jnp.float32),
            pltpu.VMEM((KCH, D144), jnp.float32),
            pltpu.VMEM_SHARED((N, D144), jnp.float32),
            pltpu.SemaphoreType.DMA,
            pltpu.SemaphoreType.DMA,
            pltpu.SemaphoreType.DMA,
        ],
        name=f"sc_layer{layer}",
        compiler_params=pltpu.CompilerParams(use_tc_tiling_on_sc=False),
    )
    def k(table_hbm, eb_hbm, idx_hbm, zeros_hbm, out_hbm,
          iring, gbuf, ebuf, mbuf, acc_sh, isem, gsem, lsem):
        c = lax.axis_index("c")
        s = lax.axis_index("s")
        wid = s * NC + c
        base = wid * RW
        irow0 = wid * NCH * 2
        pltpu.sync_copy(zeros_hbm.at[pl.ds(s * NPS, NPS)],
                        acc_sh.at[pl.ds(s * NPS, NPS)])
        plsc.subcore_barrier()

        def start(jj, b, src_row):
            pltpu.async_copy(table_hbm.at[src_row], gbuf.at[b], gsem)
            pltpu.async_copy(
                eb_hbm.at[pl.ds(base + jj * KCH, KCH), pl.ds(ECOL, 2 * C)],
                ebuf.at[b], lsem)

        def wait(b):
            pltpu.make_async_copy(table_hbm.at[pl.ds(0, KCH)], gbuf.at[b],
                                  gsem).wait()
            pltpu.make_async_copy(
                eb_hbm.at[pl.ds(0, KCH), pl.ds(ECOL, 2 * C)], ebuf.at[b],
                lsem).wait()

        pltpu.sync_copy(idx_hbm.at[pl.ds(irow0, 2 * IR)], iring.at[0])
        pltpu.async_copy(idx_hbm.at[pl.ds(irow0 + 2 * IR, 2 * IR)],
                         iring.at[1], isem)
        start(0, 0, iring.at[0, 0])
        start(1, 1, iring.at[0, 2])

        def super_step(u, _):
            slot = u % 2
            nslot = (u + 1) % 2

            @pl.when(u > 0)
            def _():
                pltpu.make_async_copy(idx_hbm.at[pl.ds(0, 2 * IR)],
                                      iring.at[0], isem).wait()

            for kk in range(IR):
                b = kk % 2
                jj = u * IR + kk
                wait(b)
                compute_chunk(gbuf.at[b], ebuf.at[b], mbuf)

                @pl.when(jj + 2 < NCH)
                def _():
                    if kk < IR - 2:
                        src_row = iring.at[slot, 2 * (kk + 2)]
                    else:
                        src_row = iring.at[nslot, 2 * (kk + 2 - IR)]
                    start(jj + 2, b, src_row)

                pltpu.sync_copy(mbuf, acc_sh.at[iring.at[slot, 2 * kk + 1]],
                                add=True)

            @pl.when(u + 2 < NSUP)
            def _():
                pltpu.async_copy(
                    idx_hbm.at[pl.ds(irow0 + (u + 2) * 2 * IR, 2 * IR)],
                    iring.at[slot], isem)
            return _

        lax.fori_loop(0, NSUP, super_step, None)
        plsc.subcore_barrier()
        pltpu.sync_copy(acc_sh.at[pl.ds(s * NPS, NPS)],
                        out_hbm.at[c, pl.ds(s * NPS, NPS)])

    return k(table, eb, idx_il, zeros_nd)


_EBLK = 2048


def _edge_dense_kernel(r2, sh16, w0, w1):
    def body(r_ref, sh_ref, w0_ref, w1_ref, eb_ref):
        rr = jnp.broadcast_to(r_ref[0], (NB, _EBLK))
        n = lax.broadcasted_iota(jnp.int32, (NB, 1), 0).astype(jnp.float32) + 1.0
        safe = jnp.where(rr > 0.0, rr, 1.0)
        rbT = jnp.sqrt(2.0 / RCUT) * jnp.sin(n * (jnp.pi / RCUT) * safe) / safe
        rbT = jnp.where(rr > 0.0, rbT, 0.0)
        dn = (((0,), (0,)), ((), ()))
        rad0 = lax.dot_general(rbT, w0_ref[...], dn,
                               preferred_element_type=jnp.float32)
        rad1 = lax.dot_general(rbT, w1_ref[...], dn,
                               preferred_element_type=jnp.float32)
        eb_ref[...] = jnp.concatenate(
            [rad0, sh_ref[...], rad1, jnp.zeros((_EBLK, 128 - 3 * C),
                                                jnp.float32)], axis=1)

    return pl.pallas_call(
        body,
        grid=(EP // _EBLK,),
        in_specs=[
            pl.BlockSpec((1, 1, _EBLK), lambda i: (i, 0, 0)),
            pl.BlockSpec((_EBLK, C), lambda i: (i, 0)),
            pl.BlockSpec((NB, C), lambda i: (0, 0)),
            pl.BlockSpec((NB, C), lambda i: (0, 0)),
        ],
        out_specs=pl.BlockSpec((_EBLK, 128), lambda i: (i, 0)),
        out_shape=jax.ShapeDtypeStruct((EP, 128), jnp.float32),
    )(r2, sh16, w0, w1)


def _node_embed_kernel(na, W_embed, W_sc0, W_sc1):
    def body(na_ref, we_ref, w0_ref, w1_ref, hs_ref, nsc0_ref, nsc1_ref):
        na = na_ref[...]
        hs_ref[...] = jnp.dot(na, we_ref[...], preferred_element_type=jnp.float32)
        nsc0_ref[...] = jnp.dot(na, w0_ref[...], preferred_element_type=jnp.float32)
        nsc1_ref[...] = jnp.dot(na, w1_ref[...], preferred_element_type=jnp.float32)

    return pl.pallas_call(
        body,
        out_shape=[
            jax.ShapeDtypeStruct((N, C), jnp.float32),
            jax.ShapeDtypeStruct((N, C), jnp.float32),
            jax.ShapeDtypeStruct((N, C), jnp.float32),
        ],
    )(na, W_embed, W_sc0, W_sc1)


def _combine_kernel(agg2, sc_a, sc_b, consts, want_h):
    S, S0, P, R, w0b, w1b, w2b = consts
    NBLK = 1000
    sca_wide = sc_a.shape[1] == D144

    def body(agg_ref, sca_ref, scb_ref, s_ref, s0_ref, p_ref, r_ref, w0_ref,
             w1_ref, w2_ref, *outs):
        agg = agg_ref[0] + agg_ref[1]
        nrmb = jnp.dot(agg * agg, s_ref[...], preferred_element_type=jnp.float32)
        a0b = jnp.dot(agg, s0_ref[...], preferred_element_type=jnp.float32)
        sca = sca_ref[...][:, 0:C] if sca_wide else sca_ref[...]
        scb = jnp.dot(sca * scb_ref[...], p_ref[...],
                      preferred_element_type=jnp.float32)
        h = (w0_ref[...] * agg + w1_ref[...] * agg * a0b
             + w2_ref[...] * agg * nrmb + scb)
        pr = jnp.sum(jnp.dot(h, r_ref[...], preferred_element_type=jnp.float32),
                     axis=0, keepdims=True)
        pr_ref = outs[-1]
        i = pl.program_id(0)

        @pl.when(i == 0)
        def _():
            pr_ref[...] = jnp.zeros((1, 8), jnp.float32)

        pr_ref[...] += pr
        if want_h:
            outs[0][...] = h

    out_shape = [jax.ShapeDtypeStruct((1, 8), jnp.float32)]
    out_specs = [pl.BlockSpec((1, 8), lambda i: (0, 0))]
    if want_h:
        out_shape = [jax.ShapeDtypeStruct((N, D144), jnp.float32)] + out_shape
        out_specs = [pl.BlockSpec((NBLK, D144), lambda i: (i, 0))] + out_specs
    return pl.pallas_call(
        body,
        grid=(N // NBLK,),
        in_specs=[
            pl.BlockSpec((2, NBLK, D144), lambda i: (0, i, 0)),
            pl.BlockSpec((NBLK, D144 if sca_wide else C), lambda i: (i, 0)),
            pl.BlockSpec((NBLK, C), lambda i: (i, 0)),
            pl.BlockSpec((D144, D144), lambda i: (0, 0)),
            pl.BlockSpec((D144, D144), lambda i: (0, 0)),
            pl.BlockSpec((C, D144), lambda i: (0, 0)),
            pl.BlockSpec((D144, 8), lambda i: (0, 0)),
            pl.BlockSpec((1, D144), lambda i: (0, 0)),
            pl.BlockSpec((1, D144), lambda i: (0, 0)),
            pl.BlockSpec((1, D144), lambda i: (0, 0)),
        ],
        out_specs=out_specs,
        out_shape=out_shape,
    )(agg2, sc_a, sc_b, S, S0, P, R, w0b, w1b, w2b)


def _layer_consts(W_prod_l, W_rs_l, W_rl2_l):
    P = jnp.tile(jnp.eye(C, dtype=jnp.float32), (1, SH))
    S = P.T @ P
    S0 = jnp.concatenate([P, jnp.zeros((D144 - C, D144), jnp.float32)], axis=0)
    P0 = jnp.concatenate([jnp.eye(C, dtype=jnp.float32),
                          jnp.zeros((C, D144 - C), jnp.float32)], axis=1)
    R = jnp.zeros((D144, 8), jnp.float32)
    R = R.at[0:C, 0].set(W_rs_l)
    for j in range(5):
        R = R.at[(4 + j) * C:(5 + j) * C, 1 + j].set(W_rl2_l)
    w0b = jnp.tile(W_prod_l[0][None, :], (1, SH))
    w1b = jnp.tile(W_prod_l[1][None, :], (1, SH))
    w2b = jnp.tile(W_prod_l[2][None, :], (1, SH))
    return S, S0, P0, R, w0b, w1b, w2b


def _forward(r, sh, na, src, dst, W_embed, W_rad, W_sc, W_prod, W_rs, W_rl2):
    pad = EP - E
    r2 = jnp.pad(r, (0, pad)).reshape(EP // _EBLK, 1, _EBLK)
    sh16 = jnp.pad(sh, ((0, pad), (0, C - SH)))
    src2d = jnp.pad(src, (0, pad)).reshape(EPR, KCH)
    dst2d = jnp.pad(dst, (0, pad)).reshape(EPR, KCH)
    idx_il = jnp.stack([src2d, dst2d], axis=1).reshape(2 * EPR, KCH)
    zeros_nd = jnp.zeros((N, D144), jnp.float32)

    eb = _edge_dense_kernel(r2, sh16, W_rad[0], W_rad[1])
    h_scalar, nsc0, nsc1 = _node_embed_kernel(na, W_embed, W_sc[0], W_sc[1])

    agg0_2 = _sc_layer(h_scalar, eb, idx_il, zeros_nd, 0)
    c0 = _layer_consts(W_prod[0], W_rs[0], W_rl2[0])
    h0, pr0 = _combine_kernel(agg0_2, h_scalar, nsc0, c0, True)

    agg1_2 = _sc_layer(h0, eb, idx_il, zeros_nd, 1)
    c1 = _layer_consts(W_prod[1], W_rs[1], W_rl2[1])
    (pr1,) = _combine_kernel(agg1_2, h0, nsc1, c1, False)

    return (pr0 + pr1)[0, :6]


def kernel(x, x_v, node_attr, edge_index, W_embed, W_rad, W_sc, W_prod, W_rs, W_rl2):
    outs = []
    for b in range(x.shape[0]):
        outs.append(_forward(x[b], x_v[b], node_attr[b],
                             edge_index[b, 0], edge_index[b, 1],
                             W_embed, W_rad, W_sc, W_prod, W_rs, W_rl2))
    return jnp.stack(outs, axis=0)
